# Initial kernel scaffold; baseline (speedup 1.0000x reference)
import jax
import jax.numpy as jnp
from jax.experimental import pallas as pl


def kernel(x, s, q, senders, receivers, params):
    raise NotImplementedError("write your pallas kernel here")


# speedup vs baseline: 3.6997x; 3.6997x over previous
import functools

import jax
import jax.numpy as jnp
from jax import lax
from jax.experimental import pallas as pl
from jax.experimental.pallas import tpu as pltpu
from jax.experimental.pallas import tpu_sc as plsc

F32 = jnp.float32

P = 1024
QN = 1024
NN = 10000
NNP = 10240
TN = 512
NT = NNP // TN
F = 96
DH = 64
FP = 128
E = 160000
NW = 32
CHUNK = 128
NCHUNK = 40
EPAD = NW * NCHUNK * CHUNK
TRASH = NNP - 1
STRIPE = NNP // 16
NEG = -1e30
GRING = 2
RING = 4


def _enc_body(x_ref, xs_ref, wg_ref, w1a_ref, w1b_ref,
              lat_ref, a_ref, b_ref, m_ref, d_ref):
    ph = pl.program_id(0)
    i = pl.program_id(1)

    @pl.when(jnp.logical_and(ph == 0, i == 0))
    def _init():
        m_ref[...] = jnp.full_like(m_ref, NEG)
        d_ref[...] = jnp.zeros_like(d_ref)

    l = jnp.dot(x_ref[...], wg_ref[...], preferred_element_type=F32)
    col = i * TN + lax.broadcasted_iota(jnp.int32, l.shape, 1)
    l = jnp.where(col < NN, l, NEG)

    @pl.when(ph == 0)
    def _pass1():
        lm = jnp.max(l, axis=1, keepdims=True)
        m_new = jnp.maximum(m_ref[...], lm)
        d_ref[...] = (d_ref[...] * jnp.exp(m_ref[...] - m_new)
                      + jnp.sum(jnp.exp(l - m_new), axis=1, keepdims=True))
        m_ref[...] = m_new

    @pl.when(ph == 1)
    def _pass2():
        p = jnp.exp(l - m_ref[...])
        xsd = xs_ref[...] / d_ref[...]
        lat = lax.dot_general(p, xsd, (((0,), (0,)), ((), ())),
                              preferred_element_type=F32)
        lat_ref[...] = lat
        a_ref[...] = jnp.dot(lat, w1a_ref[...], preferred_element_type=F32)
        b_ref[...] = jnp.dot(lat, w1b_ref[...], preferred_element_type=F32)


def _encoder(x2, xs, wg_pad, w1a, w1b):
    return pl.pallas_call(
        _enc_body,
        grid=(2, NT),
        in_specs=[
            pl.BlockSpec((P, 64), lambda ph, i: (0, 0)),
            pl.BlockSpec((P, F), lambda ph, i: (0, 0)),
            pl.BlockSpec((64, TN), lambda ph, i: (0, i)),
            pl.BlockSpec((F, DH), lambda ph, i: (0, 0)),
            pl.BlockSpec((F, DH), lambda ph, i: (0, 0)),
        ],
        out_specs=[
            pl.BlockSpec((TN, F), lambda ph, i: (i, 0)),
            pl.BlockSpec((TN, DH), lambda ph, i: (i, 0)),
            pl.BlockSpec((TN, DH), lambda ph, i: (i, 0)),
        ],
        out_shape=[
            jax.ShapeDtypeStruct((NNP, F), F32),
            jax.ShapeDtypeStruct((NNP, DH), F32),
            jax.ShapeDtypeStruct((NNP, DH), F32),
        ],
        scratch_shapes=[pltpu.VMEM((P, 1), F32), pltpu.VMEM((P, 1), F32)],
    )(x2, xs, wg_pad, w1a, w1b)


def _gather_body(a_hbm, b_hbm, ridx_hbm, sidx_hbm, g_out,
                 ridx_v, sidx_v, buf_r, buf_s, *sems):
    gsr, gss = sems[0:RING], sems[RING:2 * RING]
    wsr, wss = sems[2 * RING:3 * RING], sems[3 * RING:4 * RING]
    wid = lax.axis_index("s") * 2 + lax.axis_index("c")
    pltpu.sync_copy(ridx_hbm.at[wid], ridx_v)
    pltpu.sync_copy(sidx_hbm.at[wid], sidx_v)
    for b in range(RING):
        pltpu.async_copy(a_hbm.at[ridx_v.at[b]], buf_r.at[b], gsr[b])
        pltpu.async_copy(b_hbm.at[sidx_v.at[b]], buf_s.at[b], gss[b])

    def outer(g, carry):
        for b in range(RING):
            j = g * RING + b
            pltpu.make_async_copy(a_hbm.at[ridx_v.at[b]],
                                  buf_r.at[b], gsr[b]).wait()
            pltpu.make_async_copy(b_hbm.at[sidx_v.at[b]],
                                  buf_s.at[b], gss[b]).wait()
            wr = pltpu.async_copy(buf_r.at[b],
                                  g_out.at[wid, j, :, pl.ds(0, DH)], wsr[b])
            ws = pltpu.async_copy(buf_s.at[b],
                                  g_out.at[wid, j, :, pl.ds(DH, DH)], wss[b])
            wr.wait()
            ws.wait()

            @pl.when(j + RING < NCHUNK)
            def _next():
                pltpu.async_copy(a_hbm.at[ridx_v.at[j + RING]],
                                 buf_r.at[b], gsr[b])
                pltpu.async_copy(b_hbm.at[sidx_v.at[j + RING]],
                                 buf_s.at[b], gss[b])
        return carry

    lax.fori_loop(0, NCHUNK // RING, outer, 0)


@functools.cache
def _sc_mesh():
    return plsc.VectorSubcoreMesh(core_axis_name="c", subcore_axis_name="s")


@functools.cache
def _gather_kernel():
    return pl.kernel(
        _gather_body,
        out_type=jax.ShapeDtypeStruct((NW, NCHUNK, CHUNK, FP), F32),
        mesh=_sc_mesh(),
        compiler_params=pltpu.CompilerParams(use_tc_tiling_on_sc=False),
        scratch_types=[
            pltpu.VMEM((NCHUNK, CHUNK), jnp.int32),
            pltpu.VMEM((NCHUNK, CHUNK), jnp.int32),
            pltpu.VMEM((RING, CHUNK, DH), F32),
            pltpu.VMEM((RING, CHUNK, DH), F32),
        ] + [pltpu.SemaphoreType.DMA] * (4 * RING),
    )


def _gather(a, b, ridx, sidx):
    return _gather_kernel()(a, b, ridx, sidx)


def _edge_body(g_ref, w2_ref, w3_ref, b1_ref, b2_ref, b3_ref, out_ref):
    g = g_ref[...]
    h = jnp.maximum(g[:, :DH] + g[:, DH:] + b1_ref[...], 0.0)
    h = jnp.maximum(
        jnp.dot(h, w2_ref[...], preferred_element_type=F32) + b2_ref[...], 0.0)
    out_ref[...] = jnp.dot(h, w3_ref[...], preferred_element_type=F32) + b3_ref[...]


_EROWS = 4096


def _edge_mlp(g, w2, w3p, b1, b2, b3p):
    return pl.pallas_call(
        _edge_body,
        grid=(EPAD // _EROWS,),
        in_specs=[
            pl.BlockSpec((_EROWS, FP), lambda i: (i, 0)),
            pl.BlockSpec((DH, DH), lambda i: (0, 0)),
            pl.BlockSpec((DH, FP), lambda i: (0, 0)),
            pl.BlockSpec((1, DH), lambda i: (0, 0)),
            pl.BlockSpec((1, DH), lambda i: (0, 0)),
            pl.BlockSpec((1, FP), lambda i: (0, 0)),
        ],
        out_specs=pl.BlockSpec((_EROWS, FP), lambda i: (i, 0)),
        out_shape=jax.ShapeDtypeStruct((EPAD, FP), F32),
    )(g, w2, w3p, b1, b2, b3p)


def _scatter_body(msg_hbm, ridx_hbm, zeros_hbm, out_hbm,
                  shared, ridx_v, msg_v, *sems):
    lsem, ssem = sems[0:RING], sems[RING:2 * RING]
    cid = lax.axis_index("c")
    sid = lax.axis_index("s")
    wid = sid * 2 + cid
    pltpu.sync_copy(zeros_hbm.at[pl.ds(sid * STRIPE, STRIPE)],
                    shared.at[pl.ds(sid * STRIPE, STRIPE)])
    pltpu.sync_copy(ridx_hbm.at[wid], ridx_v)
    plsc.subcore_barrier()
    for b in range(RING):
        pltpu.async_copy(msg_hbm.at[wid, b, :, pl.ds(0, F)], msg_v.at[b],
                         lsem[b])

    def outer(g, carry):
        for b in range(RING):
            j = g * RING + b
            pltpu.make_async_copy(msg_hbm.at[wid, j, :, pl.ds(0, F)],
                                  msg_v.at[b], lsem[b]).wait()
            sc = pltpu.async_copy(msg_v.at[b],
                                  shared.at[ridx_v.at[j]], ssem[b], add=True)
            sc.wait()

            @pl.when(j + RING < NCHUNK)
            def _next():
                pltpu.async_copy(msg_hbm.at[wid, j + RING, :, pl.ds(0, F)],
                                 msg_v.at[b], lsem[b])
        return carry

    lax.fori_loop(0, NCHUNK // RING, outer, 0)
    plsc.subcore_barrier()
    pltpu.sync_copy(shared.at[pl.ds(sid * STRIPE, STRIPE)],
                    out_hbm.at[cid, pl.ds(sid * STRIPE, STRIPE)])


@functools.cache
def _scatter_kernel():
    return pl.kernel(
        _scatter_body,
        out_type=jax.ShapeDtypeStruct((2, NNP, F), F32),
        mesh=_sc_mesh(),
        compiler_params=pltpu.CompilerParams(use_tc_tiling_on_sc=False),
        scratch_types=[
            pltpu.VMEM_SHARED((NNP, F), F32),
            pltpu.VMEM((NCHUNK, CHUNK), jnp.int32),
            pltpu.VMEM((RING, CHUNK, F), F32),
        ] + [pltpu.SemaphoreType.DMA] * (2 * RING),
    )


def _scatter(msg, ridx, zeros):
    return _scatter_kernel()(msg, ridx, zeros)


def _node_body_ab(lat_ref, i0_ref, i1_ref, w1a_ref, w1b_ref, b1_ref,
                  w2_ref, b2_ref, w3_ref, b3_ref, wa2_ref, wb2_ref,
                  out_ref, a_ref, b_ref):
    lat = lat_ref[...]
    inbox = i0_ref[...] + i1_ref[...]
    h = jnp.maximum(
        jnp.dot(lat, w1a_ref[...], preferred_element_type=F32)
        + jnp.dot(inbox, w1b_ref[...], preferred_element_type=F32)
        + b1_ref[...], 0.0)
    h = jnp.maximum(
        jnp.dot(h, w2_ref[...], preferred_element_type=F32) + b2_ref[...], 0.0)
    new = lat + jnp.dot(h, w3_ref[...], preferred_element_type=F32) + b3_ref[...]
    out_ref[...] = new
    a_ref[...] = jnp.dot(new, wa2_ref[...], preferred_element_type=F32)
    b_ref[...] = jnp.dot(new, wb2_ref[...], preferred_element_type=F32)


def _node_body(lat_ref, i0_ref, i1_ref, w1a_ref, w1b_ref, b1_ref,
               w2_ref, b2_ref, w3_ref, b3_ref, out_ref):
    lat = lat_ref[...]
    inbox = i0_ref[...] + i1_ref[...]
    h = jnp.maximum(
        jnp.dot(lat, w1a_ref[...], preferred_element_type=F32)
        + jnp.dot(inbox, w1b_ref[...], preferred_element_type=F32)
        + b1_ref[...], 0.0)
    h = jnp.maximum(
        jnp.dot(h, w2_ref[...], preferred_element_type=F32) + b2_ref[...], 0.0)
    out_ref[...] = (lat + jnp.dot(h, w3_ref[...], preferred_element_type=F32)
                    + b3_ref[...])


_NROWS = 1024


def _node_mlp(lat, i0, i1, w1a, w1b, b1, w2, b2, w3, b3, wa2=None, wb2=None):
    row = pl.BlockSpec((_NROWS, F), lambda i: (i, 0))
    rowp = pl.BlockSpec((_NROWS, FP), lambda i: (i, 0))
    full = lambda r, c: pl.BlockSpec((r, c), lambda i: (0, 0))
    in_specs = [row, row, row, full(F, DH), full(F, DH), full(1, DH),
                full(DH, DH), full(1, DH), full(DH, F), full(1, F)]
    args = [lat, i0, i1, w1a, w1b, b1, w2, b2, w3, b3]
    if wa2 is not None:
        in_specs += [full(F, DH), full(F, DH)]
        args += [wa2, wb2]
        return pl.pallas_call(
            _node_body_ab,
            grid=(NNP // _NROWS,),
            in_specs=in_specs,
            out_specs=[row,
                       pl.BlockSpec((_NROWS, DH), lambda i: (i, 0)),
                       pl.BlockSpec((_NROWS, DH), lambda i: (i, 0))],
            out_shape=[jax.ShapeDtypeStruct((NNP, F), F32),
                       jax.ShapeDtypeStruct((NNP, DH), F32),
                       jax.ShapeDtypeStruct((NNP, DH), F32)],
        )(*args)
    return pl.pallas_call(
        _node_body,
        grid=(NNP // _NROWS,),
        in_specs=in_specs,
        out_specs=row,
        out_shape=jax.ShapeDtypeStruct((NNP, F), F32),
    )(*args)


def _dec_body(q_ref, wg_ref, lat_ref, w1z_ref, w1q_ref, b1_ref,
              w2_ref, b2_ref, w3_ref, b3_ref, out_ref, m_ref, d_ref, z_ref):
    i = pl.program_id(0)

    @pl.when(i == 0)
    def _init():
        m_ref[...] = jnp.full_like(m_ref, NEG)
        d_ref[...] = jnp.zeros_like(d_ref)
        z_ref[...] = jnp.zeros_like(z_ref)

    l = jnp.dot(q_ref[...], wg_ref[...], preferred_element_type=F32)
    col = i * TN + lax.broadcasted_iota(jnp.int32, l.shape, 1)
    l = jnp.where(col < NN, l, NEG)
    lm = jnp.max(l, axis=1, keepdims=True)
    m_new = jnp.maximum(m_ref[...], lm)
    scale = jnp.exp(m_ref[...] - m_new)
    p = jnp.exp(l - m_new)
    d_ref[...] = d_ref[...] * scale + jnp.sum(p, axis=1, keepdims=True)
    z_ref[...] = (z_ref[...] * scale
                  + jnp.dot(p, lat_ref[...], preferred_element_type=F32))
    m_ref[...] = m_new

    @pl.when(i == NT - 1)
    def _final():
        z = z_ref[...] / d_ref[...]
        h = jnp.maximum(
            jnp.dot(z, w1z_ref[...], preferred_element_type=F32)
            + jnp.dot(q_ref[...], w1q_ref[...], preferred_element_type=F32)
            + b1_ref[...], 0.0)
        h = jnp.maximum(
            jnp.dot(h, w2_ref[...], preferred_element_type=F32) + b2_ref[...],
            0.0)
        out_ref[...] = (jnp.dot(h, w3_ref[...], preferred_element_type=F32)
                        + b3_ref[...])


def _decoder(q2, wg_pad, lat, w1z, w1q, b1, w2, b2, w3, b3):
    return pl.pallas_call(
        _dec_body,
        grid=(NT,),
        in_specs=[
            pl.BlockSpec((QN, 64), lambda i: (0, 0)),
            pl.BlockSpec((64, TN), lambda i: (0, i)),
            pl.BlockSpec((TN, F), lambda i: (i, 0)),
            pl.BlockSpec((F, DH), lambda i: (0, 0)),
            pl.BlockSpec((64, DH), lambda i: (0, 0)),
            pl.BlockSpec((1, DH), lambda i: (0, 0)),
            pl.BlockSpec((DH, DH), lambda i: (0, 0)),
            pl.BlockSpec((1, DH), lambda i: (0, 0)),
            pl.BlockSpec((DH, 32), lambda i: (0, 0)),
            pl.BlockSpec((1, 32), lambda i: (0, 0)),
        ],
        out_specs=pl.BlockSpec((QN, 32), lambda i: (0, 0)),
        out_shape=jax.ShapeDtypeStruct((QN, 32), F32),
        scratch_shapes=[pltpu.VMEM((QN, 1), F32), pltpu.VMEM((QN, 1), F32),
                        pltpu.VMEM((QN, F), F32)],
    )(q2, wg_pad, lat, w1z, w1q, b1, w2, b2, w3, b3)


def kernel(x, s, q, senders, receivers, params):
    x2, s2, q2 = x[0], s[0], q[0]
    xs = jnp.concatenate([x2, s2], axis=1)
    wg_pad = jnp.pad(params["Wg"], ((0, 0), (0, NNP - NN)))
    ridx = jnp.pad(receivers, (0, EPAD - E),
                   constant_values=TRASH).reshape(NW, NCHUNK, CHUNK)
    sidx = jnp.pad(senders, (0, EPAD - E),
                   constant_values=TRASH).reshape(NW, NCHUNK, CHUNK)
    zeros = jnp.zeros((NNP, F), F32)

    blocks = params["blocks"]
    m0, n0 = blocks[0]["msg"], blocks[0]["node"]
    m1, n1 = blocks[1]["msg"], blocks[1]["node"]

    def rb(v):
        return v.reshape(1, -1)

    def padw(w):
        return jnp.pad(w, ((0, 0), (0, FP - F)))

    lat, a, b = _encoder(x2, xs, wg_pad, m0["W1"][:F], m0["W1"][F:])

    for bi, (mm, nn_) in enumerate(((m0, n0), (m1, n1))):
        g = _gather(a, b, ridx, sidx)
        msg = _edge_mlp(g.reshape(EPAD, FP), mm["W2"], padw(mm["W3"]),
                        rb(mm["b1"]), rb(mm["b2"]), rb(padw(mm["b3"][None])))
        inbox = _scatter(msg.reshape(NW, NCHUNK, CHUNK, FP), ridx, zeros)
        if bi == 0:
            lat, a, b = _node_mlp(lat, inbox[0], inbox[1],
                                nn_["W1"][:F], nn_["W1"][F:], rb(nn_["b1"]),
                                nn_["W2"], rb(nn_["b2"]), nn_["W3"],
                                rb(nn_["b3"]),
                                m1["W1"][:F], m1["W1"][F:])
        else:
            lat = _node_mlp(lat, inbox[0], inbox[1],
                            nn_["W1"][:F], nn_["W1"][F:], rb(nn_["b1"]),
                            nn_["W2"], rb(nn_["b2"]), nn_["W3"], rb(nn_["b3"]))

    dec = params["dec"]
    out = _decoder(q2, wg_pad, lat, dec["W1"][:F], dec["W1"][F:], rb(dec["b1"]),
                   dec["W2"], rb(dec["b2"]), dec["W3"], rb(dec["b3"]))
    return out[None]
